# baseline (speedup 1.0000x reference)
import functools

import jax
import jax.numpy as jnp
from jax import lax
from jax.experimental import pallas as pl
from jax.experimental.pallas import tpu as pltpu
from jax.experimental.pallas import tpu_sc as plsc

_B = 256
_R = 16
_NC = 2
_NS = 16
_L = 16
_NW_USED = _B // _L


def _sc_gather(t32, alpha_sq):
    mesh = plsc.VectorSubcoreMesh(core_axis_name="c", subcore_axis_name="s")

    @functools.partial(
        pl.kernel,
        out_type=jax.ShapeDtypeStruct((_B,), jnp.float32),
        mesh=mesh,
        scratch_types=[
            pltpu.VMEM((_L,), jnp.int32),
            pltpu.VMEM((1000,), jnp.float32),
            pltpu.VMEM((_L,), jnp.float32),
        ],
        compiler_params=pltpu.CompilerParams(needs_layout_passes=False),
    )
    def gather_kernel(t_hbm, tab_hbm, out_hbm, idx_v, tab_v, val_v):
        wid = lax.axis_index("s") * _NC + lax.axis_index("c")

        @pl.when(wid < _NW_USED)
        def _():
            base = wid * _L
            pltpu.sync_copy(t_hbm.at[pl.ds(base, _L)], idx_v)
            pltpu.sync_copy(tab_hbm, tab_v)
            val_v[...] = plsc.load_gather(tab_v, [idx_v[...]])
            pltpu.sync_copy(val_v, out_hbm.at[pl.ds(base, _L)])

    return gather_kernel(t32, alpha_sq)


def _tc_body(a_ref, x_ref, e_ref, o_ref):
    i = pl.program_id(0)
    row_ids = jax.lax.broadcasted_iota(jnp.int32, (_R, 1, 1, 1), 0)
    al = jnp.zeros((_R, 1, 1, 1), jnp.float32)
    sg = jnp.zeros((_R, 1, 1, 1), jnp.float32)
    for r in range(_R):
        a = a_ref[i * _R + r]
        al = jnp.where(row_ids == r, jnp.sqrt(a), al)
        sg = jnp.where(row_ids == r, jnp.sqrt(1.0 - a), sg)
    o_ref[...] = al * x_ref[...] + sg * e_ref[...]


def kernel(x, eps, t, alpha_sq):
    t32 = t.astype(jnp.int32)
    a_sel = _sc_gather(t32, alpha_sq)
    blk = (_R,) + x.shape[1:]
    grid_spec = pltpu.PrefetchScalarGridSpec(
        num_scalar_prefetch=1,
        grid=(_B // _R,),
        in_specs=[
            pl.BlockSpec(blk, lambda i, a_ref: (i, 0, 0, 0)),
            pl.BlockSpec(blk, lambda i, a_ref: (i, 0, 0, 0)),
        ],
        out_specs=pl.BlockSpec(blk, lambda i, a_ref: (i, 0, 0, 0)),
    )
    return pl.pallas_call(
        _tc_body,
        grid_spec=grid_spec,
        out_shape=jax.ShapeDtypeStruct(x.shape, x.dtype),
        compiler_params=pltpu.CompilerParams(
            vmem_limit_bytes=100 * 1024 * 1024,
        ),
    )(a_sel, x, eps)


# speedup vs baseline: 1.3242x; 1.3242x over previous
import jax
import jax.numpy as jnp
from jax.experimental import pallas as pl
from jax.experimental.pallas import tpu as pltpu

_B = 256
_R = 16


def _body(t_ref, a_ref, x_ref, e_ref, o_ref):
    i = pl.program_id(0)
    row_ids = jax.lax.broadcasted_iota(jnp.int32, (_R, 1, 1, 1), 0)
    al = jnp.zeros((_R, 1, 1, 1), jnp.float32)
    sg = jnp.zeros((_R, 1, 1, 1), jnp.float32)
    for r in range(_R):
        a = a_ref[t_ref[i * _R + r]]
        al = jnp.where(row_ids == r, jnp.sqrt(a), al)
        sg = jnp.where(row_ids == r, jnp.sqrt(1.0 - a), sg)
    o_ref[...] = al * x_ref[...] + sg * e_ref[...]


def kernel(x, eps, t, alpha_sq):
    t32 = t.astype(jnp.int32)
    blk = (_R,) + x.shape[1:]
    grid_spec = pltpu.PrefetchScalarGridSpec(
        num_scalar_prefetch=2,
        grid=(_B // _R,),
        in_specs=[
            pl.BlockSpec(blk, lambda i, t_ref, a_ref: (i, 0, 0, 0)),
            pl.BlockSpec(blk, lambda i, t_ref, a_ref: (i, 0, 0, 0)),
        ],
        out_specs=pl.BlockSpec(blk, lambda i, t_ref, a_ref: (i, 0, 0, 0)),
    )
    return pl.pallas_call(
        _body,
        grid_spec=grid_spec,
        out_shape=jax.ShapeDtypeStruct(x.shape, x.dtype),
    )(t32, alpha_sq, x, eps)
